# baseline (speedup 1.0000x reference)
import functools

import jax
import jax.numpy as jnp
from jax.experimental import pallas as pl
from jax.experimental.pallas import tpu as pltpu

_IGNORE = -100
_F8 = jnp.float8_e4m3fn


def _fused_tagger_kernel(emb_ref, labels_ref, w_ih_ref, b_ref, w_hh_ref,
                         w_cls_ref, b_cls_ref, total_ref, count_ref,
                         h_sc, c_sc, hbuf_sc, wih8_sc, whh8_sc, wcls8_sc,
                         *, t_blk, b_blk, hdim):
    @pl.when(pl.program_id(0) == 0)
    def _init():
        h_sc[...] = jnp.zeros_like(h_sc)
        c_sc[...] = jnp.zeros_like(c_sc)
        total_ref[...] = jnp.zeros_like(total_ref)
        count_ref[...] = jnp.zeros_like(count_ref)
        gcols = jax.lax.broadcasted_iota(jnp.int32, (1, 4 * hdim), 1)
        gsc = jnp.where((gcols >= 2 * hdim) & (gcols < 3 * hdim), 1.0, 0.5)
        wih8_sc[...] = (w_ih_ref[...] * gsc).astype(_F8)
        whh8_sc[...] = (w_hh_ref[...] * gsc).astype(_F8)
        wcls8_sc[...] = w_cls_ref[...].astype(_F8)

    gx = jnp.dot(emb_ref[0].astype(_F8), wih8_sc[...],
                 preferred_element_type=jnp.float32) + b_ref[...]

    whh = whh8_sc[...]
    h = h_sc[...]
    c = c_sc[...]
    for t in range(t_blk):
        gates = gx[t * b_blk:(t + 1) * b_blk] + jnp.dot(
            h, whh, preferred_element_type=jnp.float32)
        i_g = 0.5 * jnp.tanh(gates[:, 0 * hdim:1 * hdim]) + 0.5
        f_g = 0.5 * jnp.tanh(gates[:, 1 * hdim:2 * hdim]) + 0.5
        g_g = jnp.tanh(gates[:, 2 * hdim:3 * hdim])
        o_g = 0.5 * jnp.tanh(gates[:, 3 * hdim:4 * hdim]) + 0.5
        c = f_g * c + i_g * g_g
        h = (o_g * jnp.tanh(c)).astype(_F8)
        hbuf_sc[t * b_blk:(t + 1) * b_blk, :] = h
    h_sc[...] = h
    c_sc[...] = c

    logits = jnp.dot(hbuf_sc[...], wcls8_sc[...],
                     preferred_element_type=jnp.float32) + b_cls_ref[...]
    lab = labels_ref[0, 0]
    valid = lab != _IGNORE
    m = jnp.max(logits, axis=1, keepdims=True)
    lse = m + jnp.log(jnp.sum(jnp.exp(logits - m), axis=1, keepdims=True))
    cls = jax.lax.broadcasted_iota(jnp.int32, logits.shape, 1)
    safe = jnp.where(valid, lab, 0)
    picked = jnp.sum(jnp.where(cls == safe, logits, 0.0), axis=1,
                     keepdims=True)
    nll = jnp.where(valid, lse - picked, 0.0)
    total_ref[...] += jnp.sum(nll).reshape(1, 1)
    count_ref[...] += jnp.sum(valid.astype(jnp.float32)).reshape(1, 1)


def kernel(tokens, labels, embedding, w_ih_t, w_hh_t, b, w_cls_t, b_cls):
    B, T = tokens.shape
    E = embedding.shape[1]
    H = w_hh_t.shape[0]
    C = w_cls_t.shape[1]

    b_blk = B
    t_blk = 32 if T % 32 == 0 else T
    n_tb = T // t_blk
    rows = t_blk * b_blk

    tokens_r = tokens.T.reshape(T * B)
    emb_r = embedding[tokens_r].reshape(1, T * B, E)
    labels_r = labels.reshape(B, n_tb, t_blk) \
                     .transpose(1, 2, 0).reshape(1, n_tb, rows, 1)
    gate_scale = jnp.concatenate([
        jnp.full((1, H), 0.5, jnp.float32),
        jnp.full((1, H), 0.5, jnp.float32),
        jnp.ones((1, H), jnp.float32),
        jnp.full((1, H), 0.5, jnp.float32)], axis=1)

    total, count = pl.pallas_call(
        functools.partial(_fused_tagger_kernel, t_blk=t_blk, b_blk=b_blk,
                          hdim=H),
        out_shape=(jax.ShapeDtypeStruct((1, 1), jnp.float32),
                   jax.ShapeDtypeStruct((1, 1), jnp.float32)),
        grid_spec=pltpu.PrefetchScalarGridSpec(
            num_scalar_prefetch=0,
            grid=(n_tb,),
            in_specs=[
                pl.BlockSpec((1, rows, E), lambda t: (0, t, 0)),
                pl.BlockSpec((1, 1, rows, 1), lambda t: (0, t, 0, 0)),
                pl.BlockSpec((E, 4 * H), lambda t: (0, 0)),
                pl.BlockSpec((1, 4 * H), lambda t: (0, 0)),
                pl.BlockSpec((H, 4 * H), lambda t: (0, 0)),
                pl.BlockSpec((H, C), lambda t: (0, 0)),
                pl.BlockSpec((1, C), lambda t: (0, 0)),
            ],
            out_specs=[
                pl.BlockSpec((1, 1), lambda t: (0, 0)),
                pl.BlockSpec((1, 1), lambda t: (0, 0)),
            ],
            scratch_shapes=[
                pltpu.VMEM((b_blk, H), _F8),
                pltpu.VMEM((b_blk, H), jnp.float32),
                pltpu.VMEM((rows, H), _F8),
                pltpu.VMEM((E, 4 * H), _F8),
                pltpu.VMEM((H, 4 * H), _F8),
                pltpu.VMEM((H, C), _F8),
            ],
        ),
        compiler_params=pltpu.CompilerParams(
            dimension_semantics=("arbitrary",),
            vmem_limit_bytes=64 * 1024 * 1024),
    )(emb_r, labels_r, w_ih_t, b * gate_scale, w_hh_t, w_cls_t, b_cls)

    return total[0, 0] / count[0, 0]


# speedup vs baseline: 1.0110x; 1.0110x over previous
import functools

import jax
import jax.numpy as jnp
from jax.experimental import pallas as pl
from jax.experimental.pallas import tpu as pltpu

_IGNORE = -100
_F8 = jnp.float8_e4m3fn


def _fused_tagger_kernel(emb_ref, labels_ref, w_ih_ref, b_ref, w_hh_ref,
                         w_cls_ref, b_cls_ref, total_ref, count_ref,
                         h_sc, c_sc, hbuf_sc, wih8_sc, whh8_sc, wcls8_sc,
                         *, t_blk, b_blk, hdim):
    @pl.when(pl.program_id(0) == 0)
    def _init():
        h_sc[...] = jnp.zeros_like(h_sc)
        c_sc[...] = jnp.zeros_like(c_sc)
        total_ref[...] = jnp.zeros_like(total_ref)
        count_ref[...] = jnp.zeros_like(count_ref)
        gcols = jax.lax.broadcasted_iota(jnp.int32, (1, 4 * hdim), 1)
        gsc = jnp.where((gcols >= 2 * hdim) & (gcols < 3 * hdim), 1.0, 0.5)
        wih8_sc[...] = (w_ih_ref[...] * gsc).astype(_F8)
        whh8_sc[...] = (w_hh_ref[...] * gsc).astype(_F8)
        wcls8_sc[...] = w_cls_ref[...].astype(_F8)

    gx = jnp.dot(emb_ref[0].astype(_F8), wih8_sc[...],
                 preferred_element_type=jnp.float32) + b_ref[...]

    whh = whh8_sc[...]
    for t in range(t_blk):
        gates = gx[t * b_blk:(t + 1) * b_blk] + jnp.dot(
            h_sc[...], whh, preferred_element_type=jnp.float32)
        i_g = 0.5 * jnp.tanh(gates[:, 0 * hdim:1 * hdim]) + 0.5
        f_g = 0.5 * jnp.tanh(gates[:, 1 * hdim:2 * hdim]) + 0.5
        g_g = jnp.tanh(gates[:, 2 * hdim:3 * hdim])
        o_g = 0.5 * jnp.tanh(gates[:, 3 * hdim:4 * hdim]) + 0.5
        c_new = f_g * c_sc[...] + i_g * g_g
        h_new = (o_g * jnp.tanh(c_new)).astype(_F8)
        c_sc[...] = c_new
        h_sc[...] = h_new
        hbuf_sc[t * b_blk:(t + 1) * b_blk, :] = h_new

    logits = jnp.dot(hbuf_sc[...], wcls8_sc[...],
                     preferred_element_type=jnp.float32) + b_cls_ref[...]
    lab = labels_ref[0, 0]
    valid = lab != _IGNORE
    m = jnp.max(logits, axis=1, keepdims=True)
    lse = m + jnp.log(jnp.sum(jnp.exp(logits - m), axis=1, keepdims=True))
    cls = jax.lax.broadcasted_iota(jnp.int32, logits.shape, 1)
    safe = jnp.where(valid, lab, 0)
    picked = jnp.sum(jnp.where(cls == safe, logits, 0.0), axis=1,
                     keepdims=True)
    nll = jnp.where(valid, lse - picked, 0.0)
    total_ref[...] += jnp.sum(nll).reshape(1, 1)
    count_ref[...] += jnp.sum(valid.astype(jnp.float32)).reshape(1, 1)


def kernel(tokens, labels, embedding, w_ih_t, w_hh_t, b, w_cls_t, b_cls):
    B, T = tokens.shape
    E = embedding.shape[1]
    H = w_hh_t.shape[0]
    C = w_cls_t.shape[1]

    b_blk = B
    t_blk = 64 if T % 64 == 0 else T
    n_tb = T // t_blk
    rows = t_blk * b_blk

    tokens_r = tokens.T.reshape(T * B)
    emb_r = embedding[tokens_r].reshape(1, T * B, E)
    labels_r = labels.reshape(B, n_tb, t_blk) \
                     .transpose(1, 2, 0).reshape(1, n_tb, rows, 1)
    gate_scale = jnp.concatenate([
        jnp.full((1, H), 0.5, jnp.float32),
        jnp.full((1, H), 0.5, jnp.float32),
        jnp.ones((1, H), jnp.float32),
        jnp.full((1, H), 0.5, jnp.float32)], axis=1)

    total, count = pl.pallas_call(
        functools.partial(_fused_tagger_kernel, t_blk=t_blk, b_blk=b_blk,
                          hdim=H),
        out_shape=(jax.ShapeDtypeStruct((1, 1), jnp.float32),
                   jax.ShapeDtypeStruct((1, 1), jnp.float32)),
        grid_spec=pltpu.PrefetchScalarGridSpec(
            num_scalar_prefetch=0,
            grid=(n_tb,),
            in_specs=[
                pl.BlockSpec((1, rows, E), lambda t: (0, t, 0)),
                pl.BlockSpec((1, 1, rows, 1), lambda t: (0, t, 0, 0)),
                pl.BlockSpec((E, 4 * H), lambda t: (0, 0)),
                pl.BlockSpec((1, 4 * H), lambda t: (0, 0)),
                pl.BlockSpec((H, 4 * H), lambda t: (0, 0)),
                pl.BlockSpec((H, C), lambda t: (0, 0)),
                pl.BlockSpec((1, C), lambda t: (0, 0)),
            ],
            out_specs=[
                pl.BlockSpec((1, 1), lambda t: (0, 0)),
                pl.BlockSpec((1, 1), lambda t: (0, 0)),
            ],
            scratch_shapes=[
                pltpu.VMEM((b_blk, H), _F8),
                pltpu.VMEM((b_blk, H), jnp.float32),
                pltpu.VMEM((rows, H), _F8),
                pltpu.VMEM((E, 4 * H), _F8),
                pltpu.VMEM((H, 4 * H), _F8),
                pltpu.VMEM((H, C), _F8),
            ],
        ),
        compiler_params=pltpu.CompilerParams(
            dimension_semantics=("arbitrary",),
            vmem_limit_bytes=64 * 1024 * 1024),
    )(emb_r, labels_r, w_ih_t, b * gate_scale, w_hh_t, w_cls_t, b_cls)

    return total[0, 0] / count[0, 0]
